# baseline (speedup 1.0000x reference)
import functools

import jax
import jax.numpy as jnp
from jax import lax
from jax.experimental import pallas as pl
from jax.experimental.pallas import tpu as pltpu
from jax.experimental.pallas import tpu_sc as plsc

EMB_DIM = 64
EMB_NUM = 12
NPAIR = EMB_NUM * EMB_NUM
PD = 2 * EMB_DIM

NC = 2
NS = 16
NW = NC * NS

IDXW = 128
CHUNK = 128
NBUF = 4
LANES = 16
CUNROLL = 4


def _prep_body(pe_ref, gamma_ref, beta_ref, w_ref, b_ref, xa_ref, xb_ref,
               tab_ref, pidx_ref):
    n = EMB_NUM
    row = lax.broadcasted_iota(jnp.int32, (n, n), 0)
    col = lax.broadcasted_iota(jnp.int32, (n, n), 1)
    h = jnp.where(row == col, jnp.float32(1.0), jnp.float32(0.0)) + jnp.float32(0.05)
    mean = jnp.mean(h, axis=1, keepdims=True)
    var = jnp.mean((h - mean) ** 2, axis=1, keepdims=True)
    hn = (h - mean) / jnp.sqrt(var + jnp.float32(1e-5))
    hn = hn * gamma_ref[...] + beta_ref[...]
    t = lax.dot_general(hn, w_ref[...], (((1,), (1,)), ((), ())),
                        preferred_element_type=jnp.float32)
    t = jnp.maximum(t + b_ref[...], jnp.float32(0.0)) + pe_ref[...]

    p_iota = lax.broadcasted_iota(jnp.int32, (NPAIR, n), 0)
    c_iota = lax.broadcasted_iota(jnp.int32, (NPAIR, n), 1)
    sel_a = (p_iota // n == c_iota).astype(jnp.float32)
    sel_b = (p_iota % n == c_iota).astype(jnp.float32)
    tab_ref[:, :EMB_DIM] = lax.dot_general(
        sel_a, t, (((1,), (0,)), ((), ())), preferred_element_type=jnp.float32)
    tab_ref[:, EMB_DIM:] = lax.dot_general(
        sel_b, t, (((1,), (0,)), ((), ())), preferred_element_type=jnp.float32)

    pidx_ref[...] = xa_ref[...] * n + xb_ref[...]


def _prep(pe, gamma, beta, W, b, xa, xb):
    return pl.pallas_call(
        _prep_body,
        out_shape=[
            jax.ShapeDtypeStruct((NPAIR, PD), jnp.float32),
            jax.ShapeDtypeStruct(xa.shape, jnp.int32),
        ],
    )(pe, gamma.reshape(1, EMB_NUM), beta.reshape(1, EMB_NUM),
      W, b.reshape(1, EMB_DIM), xa, xb)


def _make_gather(total_pairs):
    assert total_pairs % (NW * CHUNK * NBUF) == 0
    per_w = total_pairs // NW
    n_groups = per_w // (CHUNK * NBUF)
    mesh = plsc.VectorSubcoreMesh(core_axis_name="c", subcore_axis_name="s")

    @functools.partial(
        pl.kernel,
        mesh=mesh,
        compiler_params=pltpu.CompilerParams(needs_layout_passes=False),
        out_type=jax.ShapeDtypeStruct((total_pairs * PD,), jnp.float32),
        scratch_types=[
            pltpu.VMEM((NPAIR * PD,), jnp.float32),
            pltpu.VMEM((per_w,), jnp.int32),
            [pltpu.VMEM((CHUNK * PD,), jnp.float32) for _ in range(NBUF)],
            [pltpu.SemaphoreType.DMA for _ in range(NBUF)],
        ],
    )
    def gather_kernel(table_hbm, idx_hbm, out_hbm, table_v, idx_v, rows,
                      ssems):
        sid = lax.axis_index("s")
        wid = sid * NC + lax.axis_index("c")
        base = wid * per_w
        pltpu.sync_copy(table_hbm, table_v)
        pltpu.sync_copy(idx_hbm.at[pl.ds(base, per_w)], idx_v)

        lane = lax.broadcasted_iota(jnp.int32, (LANES,), 0)
        row_bases = [(lane + r * LANES) * PD for r in range(CHUNK // LANES)]

        def group(g, carry):
            for bf in range(NBUF):
                off = (g * NBUF + bf) * CHUNK

                @pl.when(g > 0)
                def _wait_prev_store():
                    pltpu.make_async_copy(
                        rows[bf], out_hbm.at[pl.ds((base + off) * PD, CHUNK * PD)],
                        ssems[bf]).wait()

                p_addrs = [idx_v[pl.ds(off + r * LANES, LANES)] * PD
                           for r in range(CHUNK // LANES)]

                def cols(ci, carry2):
                    c0 = ci * CUNROLL
                    for cc in range(CUNROLL):
                        c_vec = jnp.full((LANES,), c0, jnp.int32) + cc
                        for r in range(CHUNK // LANES):
                            vals = plsc.load_gather(table_v,
                                                    [p_addrs[r] + c_vec])
                            plsc.store_scatter(rows[bf],
                                               [row_bases[r] + c_vec], vals)
                    return carry2

                lax.fori_loop(0, PD // CUNROLL, cols, 0)
                pltpu.async_copy(rows[bf],
                                 out_hbm.at[pl.ds((base + off) * PD, CHUNK * PD)],
                                 ssems[bf])
            return carry

        lax.fori_loop(0, n_groups, group, 0)
        for bf in range(NBUF):
            pltpu.make_async_copy(
                rows[bf], out_hbm.at[pl.ds(base * PD, CHUNK * PD)],
                ssems[bf]).wait()

    return gather_kernel


def kernel(x, pe, gamma, beta, W, b):
    Bb, Ll = x.shape
    total_pairs = (Bb * Ll) // 2
    xp = x.reshape(total_pairs, 2).astype(jnp.int32)
    xa = xp[:, 0].reshape(total_pairs // IDXW, IDXW)
    xb = xp[:, 1].reshape(total_pairs // IDXW, IDXW)
    table2, pidx = _prep(pe, gamma, beta, W, b, xa, xb)
    out = _make_gather(total_pairs)(table2.reshape(NPAIR * PD),
                                    pidx.reshape(total_pairs))
    return out.reshape(Bb, Ll, EMB_DIM)


# speedup vs baseline: 2.7943x; 2.7943x over previous
import functools

import jax
import jax.numpy as jnp
from jax import lax
from jax.experimental import pallas as pl
from jax.experimental.pallas import tpu as pltpu
from jax.experimental.pallas import tpu_sc as plsc

EMB_DIM = 64
EMB_NUM = 12
NQUAD = EMB_NUM ** 4
QD = 4 * EMB_DIM

NC = 2
NS = 16
NW = NC * NS

IDXW = 128
CHUNK = 128
NBUF = 2


def _prep_body(pe_ref, gamma_ref, beta_ref, w_ref, b_ref,
               xa_ref, xb_ref, xc_ref, xd_ref, tab_ref, qidx_ref):
    n = EMB_NUM
    row = lax.broadcasted_iota(jnp.int32, (n, n), 0)
    col = lax.broadcasted_iota(jnp.int32, (n, n), 1)
    h = jnp.where(row == col, jnp.float32(1.0), jnp.float32(0.0)) + jnp.float32(0.05)
    mean = jnp.mean(h, axis=1, keepdims=True)
    var = jnp.mean((h - mean) ** 2, axis=1, keepdims=True)
    hn = (h - mean) / jnp.sqrt(var + jnp.float32(1e-5))
    hn = hn * gamma_ref[...] + beta_ref[...]
    t = lax.dot_general(hn, w_ref[...], (((1,), (1,)), ((), ())),
                        preferred_element_type=jnp.float32)
    t = jnp.maximum(t + b_ref[...], jnp.float32(0.0)) + pe_ref[...]

    q_iota = lax.broadcasted_iota(jnp.int32, (NQUAD, n), 0)
    c_iota = lax.broadcasted_iota(jnp.int32, (NQUAD, n), 1)
    for k, div in enumerate((n ** 3, n ** 2, n, 1)):
        sel = ((q_iota // div) % n == c_iota).astype(jnp.float32)
        tab_ref[:, k * EMB_DIM:(k + 1) * EMB_DIM] = lax.dot_general(
            sel, t, (((1,), (0,)), ((), ())), preferred_element_type=jnp.float32)

    qidx_ref[...] = ((xa_ref[...] * n + xb_ref[...]) * n + xc_ref[...]) * n \
        + xd_ref[...]


def _prep(pe, gamma, beta, W, b, xa, xb, xc, xd):
    return pl.pallas_call(
        _prep_body,
        out_shape=[
            jax.ShapeDtypeStruct((NQUAD, QD), jnp.float32),
            jax.ShapeDtypeStruct(xa.shape, jnp.int32),
        ],
    )(pe, gamma.reshape(1, EMB_NUM), beta.reshape(1, EMB_NUM),
      W, b.reshape(1, EMB_DIM), xa, xb, xc, xd)


def _make_gather(total_quads):
    assert total_quads % (NW * CHUNK * NBUF) == 0
    per_w = total_quads // NW
    n_groups = per_w // (CHUNK * NBUF)
    mesh = plsc.VectorSubcoreMesh(core_axis_name="c", subcore_axis_name="s")

    @functools.partial(
        pl.kernel,
        mesh=mesh,
        out_type=jax.ShapeDtypeStruct((total_quads, QD), jnp.float32),
        scratch_types=[
            pltpu.VMEM((per_w,), jnp.int32),
            [pltpu.VMEM((CHUNK, QD), jnp.float32) for _ in range(NBUF)],
            [pltpu.SemaphoreType.DMA for _ in range(NBUF)],
            [pltpu.SemaphoreType.DMA for _ in range(NBUF)],
        ],
    )
    def gather_kernel(table_hbm, idx_hbm, out_hbm, idx_v, rows, gsems, ssems):
        sid = lax.axis_index("s")
        wid = sid * NC + lax.axis_index("c")
        base = wid * per_w
        pltpu.sync_copy(idx_hbm.at[pl.ds(base, per_w)], idx_v)

        def group(g, carry):
            for bf in range(NBUF):
                off = (g * NBUF + bf) * CHUNK

                @pl.when(g > 0)
                def _wait_prev_store():
                    pltpu.make_async_copy(
                        rows[bf], out_hbm.at[pl.ds(base + off, CHUNK)],
                        ssems[bf]).wait()

                pltpu.async_copy(
                    table_hbm.at[idx_v.at[pl.ds(off, CHUNK)]],
                    rows[bf], gsems[bf])
            for bf in range(NBUF):
                off = (g * NBUF + bf) * CHUNK
                pltpu.make_async_copy(
                    table_hbm.at[idx_v.at[pl.ds(off, CHUNK)]],
                    rows[bf], gsems[bf]).wait()
                pltpu.async_copy(rows[bf], out_hbm.at[pl.ds(base + off, CHUNK)],
                                 ssems[bf])
            return carry

        lax.fori_loop(0, n_groups, group, 0)
        for bf in range(NBUF):
            pltpu.make_async_copy(
                rows[bf], out_hbm.at[pl.ds(base, CHUNK)], ssems[bf]).wait()

    return gather_kernel


def kernel(x, pe, gamma, beta, W, b):
    Bb, Ll = x.shape
    total_quads = (Bb * Ll) // 4
    xq = x.reshape(total_quads, 4).astype(jnp.int32)
    parts = [xq[:, k].reshape(total_quads // IDXW, IDXW) for k in range(4)]
    table4, qidx = _prep(pe, gamma, beta, W, b, *parts)
    out = _make_gather(total_quads)(table4, qidx.reshape(total_quads))
    return out.reshape(Bb, Ll, EMB_DIM)
